# baseline (speedup 1.0000x reference)
import numpy as np

import jax
import jax.numpy as jnp
from jax import lax
from jax.experimental import pallas as pl
from jax.experimental.pallas import tpu as pltpu

_D = 2048
_HD = (13, 128, 128)
_B = 64
_NEG = -1e9
_KB = 1024
_NK = 2


def _tf2x32(k1, k2, c1, c2):
    rot = [np.uint32(r) for r in (13, 15, 26, 6, 17, 29, 16, 24)]

    def rotl(x, d):
        return (x << d) | (x >> np.uint32(32 - d))

    ks0, ks1 = np.uint32(k1), np.uint32(k2)
    ks2 = ks0 ^ ks1 ^ np.uint32(0x1BD11BDA)
    x0 = (c1 + ks0).astype(np.uint32)
    x1 = (c2 + ks1).astype(np.uint32)
    ks = [ks0, ks1, ks2]
    rsets = [rot[0:4], rot[4:8]]
    with np.errstate(over="ignore"):
        for i in range(5):
            for r in rsets[i % 2]:
                x0 = (x0 + x1).astype(np.uint32)
                x1 = rotl(x1, r)
                x1 = x1 ^ x0
            x0 = (x0 + ks[(i + 1) % 3]).astype(np.uint32)
            x1 = (x1 + ks[(i + 2) % 3] + np.uint32(i + 1)).astype(np.uint32)
    return x0, x1


def _gumbel_np(head):
    k = _tf2x32(0, 42, np.uint32([0]), np.uint32([head]))
    size = _B * _HD[head]
    idx = np.arange(size, dtype=np.uint64)
    c1 = (idx >> np.uint64(32)).astype(np.uint32)
    c2 = (idx & np.uint64(0xFFFFFFFF)).astype(np.uint32)
    b1, b2 = _tf2x32(k[0][0], k[1][0], c1, c2)
    f = (((b1 ^ b2) >> np.uint32(9)) | np.uint32(0x3F800000)).view(np.float32)
    f = f - np.float32(1.0)
    tiny = np.float32(np.finfo(np.float32).tiny)
    u = np.maximum(tiny, f * (np.float32(1.0) - tiny) + tiny)
    return (-np.log(-np.log(u))).reshape(_B, _HD[head]).astype(np.float32)


_G = tuple(_gumbel_np(i) for i in range(3))


def _head_stats(lm, g):
    d = lm.shape[1]
    col = jax.lax.broadcasted_iota(jnp.int32, lm.shape, 1)
    z = lm + g
    zmax = jnp.max(z, axis=1, keepdims=True)
    a = jnp.min(jnp.where(z >= zmax, col, d), axis=1, keepdims=True)
    onehot = (col == a).astype(jnp.float32)
    mx = jnp.max(lm, axis=1, keepdims=True)
    e = jnp.exp(lm - mx)
    s = jnp.sum(e, axis=1, keepdims=True)
    lse = mx + jnp.log(s)
    lp_vec = lm - lse
    lp = jnp.sum(onehot * lp_vec, axis=1, keepdims=True)
    ent = -jnp.sum((e / s) * lp_vec, axis=1, keepdims=True)
    return onehot, lp, ent


def _body(x_ref, w0t_ref, w1_ref, w2_ref, m0t_ref, m1_ref, m2_ref,
          b0_ref, b1_ref, b2_ref, g0_ref, g1_ref, g2_ref, out_ref,
          y0_acc, y1_acc, y2_acc):
    k = pl.program_id(0)
    x = x_ref[:]

    @pl.when(k < _NK)
    def _accumulate():
        p0 = lax.dot_general(x, w0t_ref[:], (((1,), (1,)), ((), ())),
                             preferred_element_type=jnp.float32)
        p1 = jnp.dot(x, w1_ref[:], preferred_element_type=jnp.float32)
        p2 = jnp.dot(x, w2_ref[:], preferred_element_type=jnp.float32)

        @pl.when(k == 0)
        def _init():
            y0_acc[:] = p0
            y1_acc[:] = p1
            y2_acc[:] = p2

        @pl.when(k > 0)
        def _add():
            y0_acc[:] += p0
            y1_acc[:] += p1
            y2_acc[:] += p2

    @pl.when(k == _NK)
    def _finish():
        m0 = m0t_ref[:].T
        y0 = y0_acc[:] + b0_ref[:][None, :]
        lm0 = jnp.where(m0 > 0, y0, _NEG)
        oh0, lp0, ent0 = _head_stats(lm0, g0_ref[:])

        w1_tail = w1_ref[pl.ds(0, _HD[0]), :]
        y1 = (y1_acc[:] + b1_ref[:][None, :]
              + jnp.dot(oh0, w1_tail, preferred_element_type=jnp.float32))
        lm1 = jnp.where(m1_ref[:] > 0, y1, _NEG)
        oh1, lp1, ent1 = _head_stats(lm1, g1_ref[:])

        w2_tail = w2_ref[pl.ds(0, _HD[0] + _HD[1]), :]
        ohx = jnp.concatenate([oh0, oh1], axis=1)
        y2 = (y2_acc[:] + b2_ref[:][None, :]
              + jnp.dot(ohx, w2_tail, preferred_element_type=jnp.float32))
        lm2 = jnp.where(m2_ref[:] > 0, y2, _NEG)
        _, lp2, ent2 = _head_stats(lm2, g2_ref[:])

        lp = lp0 + lp1 + lp2
        ent = ent0 + ent1 + ent2
        colx = jax.lax.broadcasted_iota(jnp.int32, (_B, 128), 1)
        out_ref[:] = jnp.where(colx == 0, lp, jnp.where(colx == 1, ent, 0.0))


def kernel(main_input, mask0, mask1, mask2, W0, b0, W1, b1, W2, b2):
    last = _NK - 1
    grid = (_NK + 1,)
    out = pl.pallas_call(
        _body,
        grid=grid,
        in_specs=[
            pl.BlockSpec((_B, _KB), lambda k: (0, jnp.minimum(k, last))),
            pl.BlockSpec((_HD[0], _KB), lambda k: (0, jnp.minimum(k, last))),
            pl.BlockSpec((_KB, _HD[1]), lambda k: (k, 0)),
            pl.BlockSpec((_KB, _HD[2]), lambda k: (k, 0)),
            pl.BlockSpec((_HD[0], _B), lambda k: (0, 0)),
            pl.BlockSpec((_B, _HD[1]), lambda k: (0, 0)),
            pl.BlockSpec((_B, _HD[2]), lambda k: (0, 0)),
            pl.BlockSpec((_HD[0],), lambda k: (0,)),
            pl.BlockSpec((_HD[1],), lambda k: (0,)),
            pl.BlockSpec((_HD[2],), lambda k: (0,)),
            pl.BlockSpec((_B, _HD[0]), lambda k: (0, 0)),
            pl.BlockSpec((_B, _HD[1]), lambda k: (0, 0)),
            pl.BlockSpec((_B, _HD[2]), lambda k: (0, 0)),
        ],
        out_specs=pl.BlockSpec((_B, 128), lambda k: (0, 0)),
        out_shape=jax.ShapeDtypeStruct((_B, 128), jnp.float32),
        scratch_shapes=[
            pltpu.VMEM((_B, _HD[0]), jnp.float32),
            pltpu.VMEM((_B, _HD[1]), jnp.float32),
            pltpu.VMEM((_B, _HD[2]), jnp.float32),
        ],
        compiler_params=pltpu.CompilerParams(
            dimension_semantics=("arbitrary",),
        ),
    )(main_input, W0.T, W1, W2, mask0.T, mask1, mask2, b0, b1, b2,
      jnp.asarray(_G[0]), jnp.asarray(_G[1]), jnp.asarray(_G[2]))
    return out[:, :2]


# speedup vs baseline: 1.0303x; 1.0303x over previous
import numpy as np

import jax
import jax.numpy as jnp
from jax import lax
from jax.experimental import pallas as pl

_D = 2048
_HD = (13, 128, 128)
_B = 64
_NEG = -1e9


def _tf2x32(k1, k2, c1, c2):
    rot = [np.uint32(r) for r in (13, 15, 26, 6, 17, 29, 16, 24)]

    def rotl(x, d):
        return (x << d) | (x >> np.uint32(32 - d))

    ks0, ks1 = np.uint32(k1), np.uint32(k2)
    ks2 = ks0 ^ ks1 ^ np.uint32(0x1BD11BDA)
    x0 = (c1 + ks0).astype(np.uint32)
    x1 = (c2 + ks1).astype(np.uint32)
    ks = [ks0, ks1, ks2]
    rsets = [rot[0:4], rot[4:8]]
    with np.errstate(over="ignore"):
        for i in range(5):
            for r in rsets[i % 2]:
                x0 = (x0 + x1).astype(np.uint32)
                x1 = rotl(x1, r)
                x1 = x1 ^ x0
            x0 = (x0 + ks[(i + 1) % 3]).astype(np.uint32)
            x1 = (x1 + ks[(i + 2) % 3] + np.uint32(i + 1)).astype(np.uint32)
    return x0, x1


def _gumbel_np(head):
    k = _tf2x32(0, 42, np.uint32([0]), np.uint32([head]))
    size = _B * _HD[head]
    idx = np.arange(size, dtype=np.uint64)
    c1 = (idx >> np.uint64(32)).astype(np.uint32)
    c2 = (idx & np.uint64(0xFFFFFFFF)).astype(np.uint32)
    b1, b2 = _tf2x32(k[0][0], k[1][0], c1, c2)
    f = (((b1 ^ b2) >> np.uint32(9)) | np.uint32(0x3F800000)).view(np.float32)
    f = f - np.float32(1.0)
    tiny = np.float32(np.finfo(np.float32).tiny)
    u = np.maximum(tiny, f * (np.float32(1.0) - tiny) + tiny)
    return (-np.log(-np.log(u))).reshape(_B, _HD[head]).astype(np.float32)


_G = tuple(_gumbel_np(i) for i in range(3))


def _head_stats(lm, g):
    d = lm.shape[1]
    col = jax.lax.broadcasted_iota(jnp.int32, lm.shape, 1)
    z = lm + g
    zmax = jnp.max(z, axis=1, keepdims=True)
    a = jnp.min(jnp.where(z >= zmax, col, d), axis=1, keepdims=True)
    onehot = (col == a).astype(jnp.float32)
    mx = jnp.max(lm, axis=1, keepdims=True)
    e = jnp.exp(lm - mx)
    s = jnp.sum(e, axis=1, keepdims=True)
    lse = mx + jnp.log(s)
    lp_vec = lm - lse
    lp = jnp.sum(onehot * lp_vec, axis=1, keepdims=True)
    ent = -jnp.sum((e / s) * lp_vec, axis=1, keepdims=True)
    return onehot, lp, ent


def _body(x_ref, w0t_ref, w1_ref, w2_ref, m0t_ref, m1_ref, m2_ref,
          b0_ref, b1_ref, b2_ref, g0_ref, g1_ref, g2_ref, out_ref):
    x = x_ref[:]

    y0 = (lax.dot_general(x, w0t_ref[:], (((1,), (1,)), ((), ())),
                          preferred_element_type=jnp.float32)
          + b0_ref[:][None, :])
    lm0 = jnp.where(m0t_ref[:].T > 0, y0, _NEG)
    oh0, lp0, ent0 = _head_stats(lm0, g0_ref[:])

    y1 = (jnp.dot(x, w1_ref[pl.ds(0, _D), :], preferred_element_type=jnp.float32)
          + jnp.dot(oh0, w1_ref[pl.ds(_D, _HD[0]), :],
                    preferred_element_type=jnp.float32)
          + b1_ref[:][None, :])
    lm1 = jnp.where(m1_ref[:] > 0, y1, _NEG)
    oh1, lp1, ent1 = _head_stats(lm1, g1_ref[:])

    ohx = jnp.concatenate([oh0, oh1], axis=1)
    y2 = (jnp.dot(x, w2_ref[pl.ds(0, _D), :], preferred_element_type=jnp.float32)
          + jnp.dot(ohx, w2_ref[pl.ds(_D, _HD[0] + _HD[1]), :],
                    preferred_element_type=jnp.float32)
          + b2_ref[:][None, :])
    lm2 = jnp.where(m2_ref[:] > 0, y2, _NEG)
    _, lp2, ent2 = _head_stats(lm2, g2_ref[:])

    lp = lp0 + lp1 + lp2
    ent = ent0 + ent1 + ent2
    col = jax.lax.broadcasted_iota(jnp.int32, (_B, 128), 1)
    out_ref[:] = jnp.where(col == 0, lp, jnp.where(col == 64, ent, 0.0))


def kernel(main_input, mask0, mask1, mask2, W0, b0, W1, b1, W2, b2):
    out = pl.pallas_call(
        _body,
        out_shape=jax.ShapeDtypeStruct((_B, 128), jnp.float32),
    )(main_input, W0.T, W1, W2, mask0.T, mask1, mask2, b0, b1, b2,
      jnp.asarray(_G[0]), jnp.asarray(_G[1]), jnp.asarray(_G[2]))
    return jnp.concatenate([out[:, :1], out[:, 64:65]], axis=1)


# speedup vs baseline: 1.0911x; 1.0590x over previous
import numpy as np

import jax
import jax.numpy as jnp
from jax import lax
from jax.experimental import pallas as pl

_D = 2048
_HD = (13, 128, 128)
_B = 64
_NEG = -1e9


def _tf2x32(k1, k2, c1, c2):
    rot = [np.uint32(r) for r in (13, 15, 26, 6, 17, 29, 16, 24)]

    def rotl(x, d):
        return (x << d) | (x >> np.uint32(32 - d))

    ks0, ks1 = np.uint32(k1), np.uint32(k2)
    ks2 = ks0 ^ ks1 ^ np.uint32(0x1BD11BDA)
    x0 = (c1 + ks0).astype(np.uint32)
    x1 = (c2 + ks1).astype(np.uint32)
    ks = [ks0, ks1, ks2]
    rsets = [rot[0:4], rot[4:8]]
    with np.errstate(over="ignore"):
        for i in range(5):
            for r in rsets[i % 2]:
                x0 = (x0 + x1).astype(np.uint32)
                x1 = rotl(x1, r)
                x1 = x1 ^ x0
            x0 = (x0 + ks[(i + 1) % 3]).astype(np.uint32)
            x1 = (x1 + ks[(i + 2) % 3] + np.uint32(i + 1)).astype(np.uint32)
    return x0, x1


def _gumbel_np(head):
    k = _tf2x32(0, 42, np.uint32([0]), np.uint32([head]))
    size = _B * _HD[head]
    idx = np.arange(size, dtype=np.uint64)
    c1 = (idx >> np.uint64(32)).astype(np.uint32)
    c2 = (idx & np.uint64(0xFFFFFFFF)).astype(np.uint32)
    b1, b2 = _tf2x32(k[0][0], k[1][0], c1, c2)
    f = (((b1 ^ b2) >> np.uint32(9)) | np.uint32(0x3F800000)).view(np.float32)
    f = f - np.float32(1.0)
    tiny = np.float32(np.finfo(np.float32).tiny)
    u = np.maximum(tiny, f * (np.float32(1.0) - tiny) + tiny)
    return (-np.log(-np.log(u))).reshape(_B, _HD[head]).astype(np.float32)


_G = tuple(_gumbel_np(i) for i in range(3))


def _head_stats(lm, g):
    d = lm.shape[1]
    col = jax.lax.broadcasted_iota(jnp.int32, lm.shape, 1)
    z = lm + g
    zmax = jnp.max(z, axis=1, keepdims=True)
    a = jnp.min(jnp.where(z >= zmax, col, d), axis=1, keepdims=True)
    onehot = (col == a).astype(jnp.float32)
    mx = jnp.max(lm, axis=1, keepdims=True)
    e = jnp.exp(lm - mx)
    s = jnp.sum(e, axis=1, keepdims=True)
    lse = mx + jnp.log(s)
    lp_vec = lm - lse
    lp = jnp.sum(onehot * lp_vec, axis=1, keepdims=True)
    ent = -jnp.sum((e / s) * lp_vec, axis=1, keepdims=True)
    return onehot, lp, ent


def _body(x_ref, w0t_ref, w1_ref, w2_ref, m0t_ref, m1_ref, m2_ref,
          b0_ref, b1_ref, b2_ref, g0_ref, g1_ref, g2_ref, out_ref):
    x = x_ref[:]

    y0 = (lax.dot_general(x, w0t_ref[:], (((1,), (1,)), ((), ())),
                          preferred_element_type=jnp.float32)
          + b0_ref[:][None, :])
    lm0 = jnp.where(m0t_ref[:].T > 0, y0, _NEG)
    oh0, lp0, ent0 = _head_stats(lm0, g0_ref[:])

    y1 = (jnp.dot(x, w1_ref[pl.ds(0, _D), :], preferred_element_type=jnp.float32)
          + jnp.dot(oh0, w1_ref[pl.ds(_D, _HD[0]), :],
                    preferred_element_type=jnp.float32)
          + b1_ref[:][None, :])
    lm1 = jnp.where(m1_ref[:] > 0, y1, _NEG)
    oh1, lp1, ent1 = _head_stats(lm1, g1_ref[:])

    ohx = jnp.concatenate([oh0, oh1], axis=1)
    y2 = (jnp.dot(x, w2_ref[pl.ds(0, _D), :], preferred_element_type=jnp.float32)
          + jnp.dot(ohx, w2_ref[pl.ds(_D, _HD[0] + _HD[1]), :],
                    preferred_element_type=jnp.float32)
          + b2_ref[:][None, :])
    lm2 = jnp.where(m2_ref[:] > 0, y2, _NEG)
    _, lp2, ent2 = _head_stats(lm2, g2_ref[:])

    lp = lp0 + lp1 + lp2
    ent = ent0 + ent1 + ent2
    col = jax.lax.broadcasted_iota(jnp.int32, (_B, 128), 1)
    out_ref[:] = jnp.where(col == 0, lp, jnp.where(col == 1, ent, 0.0))


def kernel(main_input, mask0, mask1, mask2, W0, b0, W1, b1, W2, b2):
    out = pl.pallas_call(
        _body,
        out_shape=jax.ShapeDtypeStruct((_B, 128), jnp.float32),
    )(main_input, W0.T, W1, W2, mask0.T, mask1, mask2, b0, b1, b2,
      jnp.asarray(_G[0]), jnp.asarray(_G[1]), jnp.asarray(_G[2]))
    return out[:, :2]
